# baseline (speedup 1.0000x reference)
import functools

import jax
import jax.numpy as jnp
from jax import lax
from jax.experimental import pallas as pl
from jax.experimental.pallas import tpu as pltpu
from jax.experimental.pallas import tpu_sc as plsc

E = 1_600_000
S = 50_000
F = 16
NW = 32
PS = 1_568
S_PAD = NW * PS
C = 4_096
LOG2C = 12
BR = 2_000
NBLK = (E // 8) // BR


def _pred_body(x_ref, w_ref, o_ref):
    o_ref[...] = lax.dot_general(
        x_ref[...], w_ref[...], (((1,), (0,)), ((), ())),
        preferred_element_type=jnp.float32).astype(jnp.int32)


def _sc_body(tin_hbm, tout_hbm, seg_hbm, pred_hbm, nrate_hbm, bounds_hbm,
             out_hbm, tout_v, num_v, den_v, seg_v, tin_v, pred_v, nrate_v,
             bounds_v, sem0, sem1):
    wid = lax.axis_index("c") * 16 + lax.axis_index("s")
    seg_base = wid * PS

    pltpu.sync_copy(bounds_hbm, bounds_v)
    pltpu.sync_copy(nrate_hbm, nrate_v)
    pltpu.sync_copy(tout_hbm.at[pl.ds(seg_base, PS)], tout_v.at[pl.ds(0, PS)])
    tout_v[pl.ds(PS, 16)] = jnp.zeros((16,), jnp.float32)

    zeros16 = jnp.zeros((16,), jnp.float32)

    def zero_body(i, _):
        num_v[pl.ds(i * 16, 16)] = zeros16
        den_v[pl.ds(i * 16, 16)] = zeros16
        return 0
    lax.fori_loop(0, PS + 1, zero_body, 0)

    bvec = bounds_v[pl.ds(wid, 16)]
    a = bvec[0]
    a_end = bvec[1]
    b = lax.bitwise_and(a, -8)
    nk = lax.shift_right_logical(a_end - b + (C - 1), LOG2C)

    nrate = nrate_v[...]
    iota16 = lax.broadcasted_iota(jnp.int32, (16,), 0)
    segb_splat = jnp.full((16,), seg_base, jnp.int32)
    ps_splat = jnp.full((16,), PS, jnp.int32)
    aend_splat = jnp.full((16,), a_end, jnp.int32)
    ones16f = jnp.ones((16,), jnp.float32)
    step16 = jnp.full((16,), 16, jnp.int32)

    def dma_start(k, slot):
        start = b + lax.shift_left(k, LOG2C)
        e0 = pl.multiple_of(jnp.minimum(start, E - C), 8)
        sb = slot * C
        sem = sem0 if slot == 0 else sem1
        pltpu.async_copy(seg_hbm.at[pl.ds(e0, C)], seg_v.at[pl.ds(sb, C)],
                         sem)
        pltpu.async_copy(tin_hbm.at[pl.ds(e0, C)], tin_v.at[pl.ds(sb, C)],
                         sem)
        pltpu.async_copy(pred_hbm.at[pl.ds(e0, C)], pred_v.at[pl.ds(sb, C)],
                         sem)

    def dma_wait(slot):
        sb = slot * C
        sem = sem0 if slot == 0 else sem1
        pltpu.make_async_copy(seg_hbm.at[pl.ds(0, C)],
                              seg_v.at[pl.ds(sb, C)], sem).wait()
        pltpu.make_async_copy(tin_hbm.at[pl.ds(0, C)],
                              tin_v.at[pl.ds(sb, C)], sem).wait()
        pltpu.make_async_copy(pred_hbm.at[pl.ds(0, C)],
                              pred_v.at[pl.ds(sb, C)], sem).wait()

    def compute(k, slot):
        start = b + lax.shift_left(k, LOG2C)
        e0 = jnp.minimum(start, E - C)
        lo = jnp.maximum(a, start)
        lo_splat = jnp.full((16,), lo, jnp.int32)
        sb = slot * C
        gvv0 = jnp.full((16,), e0, jnp.int32) + iota16

        def group_body(g, gvv):
            gbase = sb + lax.shift_left(g, 4)
            segv = seg_v[pl.ds(gbase, 16)]
            tinv = tin_v[pl.ds(gbase, 16)]
            predv = pred_v[pl.ds(gbase, 16)]
            validv = jnp.logical_and(gvv >= lo_splat, gvv < aend_splat)
            slv = jnp.where(validv, segv - segb_splat, ps_splat)
            toutv = plsc.load_gather(tout_v, [slv])
            dtv = toutv - tinv
            ratev = plsc.load_gather(nrate_v, [predv])
            valv = jnp.exp(ratev * dtv)
            idxv = lax.shift_left(slv, 4) + predv
            plsc.addupdate_scatter(num_v, [idxv], valv, mask=validv)
            plsc.addupdate_scatter(den_v, [idxv], ones16f, mask=validv)
            return gvv + step16
        lax.fori_loop(0, C // 16, group_body, gvv0)

    @pl.when(nk > 0)
    def _():
        dma_start(0, 0)

    def pair_body(p, _):
        k0 = lax.shift_left(p, 1)
        k1 = k0 + 1

        @pl.when(k1 < nk)
        def _():
            dma_start(k1, 1)
        dma_wait(0)
        compute(k0, 0)

        @pl.when(k1 + 1 < nk)
        def _():
            dma_start(k1 + 1, 0)

        @pl.when(k1 < nk)
        def _():
            dma_wait(1)
            compute(k1, 1)
        return 0
    lax.fori_loop(0, lax.shift_right_logical(nk + 1, 1), pair_body, 0)

    def div_body(i, _):
        o = i * 16
        num_v[pl.ds(o, 16)] = num_v[pl.ds(o, 16)] / jnp.maximum(
            den_v[pl.ds(o, 16)], ones16f)
        return 0
    lax.fori_loop(0, PS, div_body, 0)

    pltpu.sync_copy(num_v.at[pl.ds(0, PS * F)],
                    out_hbm.at[pl.ds(seg_base * F, PS * F)])


@jax.jit
def _run(times_in, tout_pad, segment_filter_ids, oh2d, w, nrate, bounds):
    pred = pl.pallas_call(
        _pred_body,
        grid=(NBLK,),
        in_specs=[pl.BlockSpec((BR, 128), lambda i: (i, 0)),
                  pl.BlockSpec((128, 8), lambda i: (0, 0))],
        out_specs=pl.BlockSpec((BR, 8), lambda i: (i, 0)),
        out_shape=jax.ShapeDtypeStruct((E // 8, 8), jnp.int32),
    )(oh2d, w).reshape(E)

    mesh = plsc.VectorSubcoreMesh(core_axis_name="c", subcore_axis_name="s")
    f = pl.kernel(
        _sc_body,
        out_type=jax.ShapeDtypeStruct((S_PAD * F,), jnp.float32),
        mesh=mesh,
        scratch_types=[
            pltpu.VMEM((PS + 16,), jnp.float32),
            pltpu.VMEM(((PS + 1) * F,), jnp.float32),
            pltpu.VMEM(((PS + 1) * F,), jnp.float32),
            pltpu.VMEM((2 * C,), jnp.int32),
            pltpu.VMEM((2 * C,), jnp.float32),
            pltpu.VMEM((2 * C,), jnp.int32),
            pltpu.VMEM((16,), jnp.float32),
            pltpu.VMEM((48,), jnp.int32),
            pltpu.SemaphoreType.DMA,
            pltpu.SemaphoreType.DMA,
        ],
        compiler_params=pltpu.CompilerParams(needs_layout_passes=False),
    )
    return f(times_in, tout_pad, segment_filter_ids, pred, nrate, bounds)


def kernel(times_in, times_out, segment_filter_ids, one_hot_predecessor_ids,
           decay_rate):
    nrate = -jax.nn.softplus(decay_rate)
    tout_pad = jnp.pad(times_out, (0, S_PAD - S))
    limits = jnp.minimum(jnp.arange(NW + 1, dtype=jnp.int32) * PS, S)
    bounds = jnp.searchsorted(segment_filter_ids, limits, side="left",
                              method="scan_unrolled").astype(jnp.int32)
    bounds = jnp.pad(bounds, (0, 48 - (NW + 1)))
    oh2d = one_hot_predecessor_ids.reshape(E // 8, 128)
    j = jnp.arange(128)
    k = jnp.arange(8)
    w = jnp.where((j[:, None] // 16) == k[None, :],
                  (j % 16)[:, None], 0).astype(jnp.float32)
    out = _run(times_in, tout_pad, segment_filter_ids, oh2d, w, nrate, bounds)
    return out.reshape(S_PAD, F)[:S]


# speedup vs baseline: 3.2695x; 3.2695x over previous
import functools

import jax
import jax.numpy as jnp
from jax import lax
from jax.experimental import pallas as pl
from jax.experimental.pallas import tpu as pltpu
from jax.experimental.pallas import tpu_sc as plsc

E = 1_600_000
S = 50_000
F = 16
NW = 32
PS = 1_568
PST = PS + 16
S_PAD = NW * PS
C = 4_096
LOG2C = 12
BLKE = 12_800
NBLKE = E // BLKE


def _pred_body(x_ref, o_ref):
    x = x_ref[...]
    w = lax.broadcasted_iota(jnp.int32, (F, 1), 0).astype(jnp.float32)
    i = pl.program_id(0)
    o_ref[pl.ds(i * BLKE, BLKE)] = jnp.sum(x * w, axis=0).astype(jnp.int32)


def _sc_body(tin_hbm, tout_hbm, seg_hbm, pred_hbm, nrate_hbm, bounds_hbm,
             out_hbm, tout_v, num_v, den_v, seg_v, tin_v, pred_v, nrate_v,
             bounds_v, sem0, sem1):
    wid = lax.axis_index("c") * 16 + lax.axis_index("s")
    seg_base = wid * PS

    pltpu.sync_copy(bounds_hbm, bounds_v)
    pltpu.sync_copy(nrate_hbm, nrate_v)
    pltpu.sync_copy(tout_hbm.at[pl.ds(seg_base, PS)], tout_v.at[pl.ds(0, PS)])
    tout_v[pl.ds(PS, 16)] = jnp.zeros((16,), jnp.float32)

    zeros16 = jnp.zeros((16,), jnp.float32)

    def zero_body(i, _):
        num_v[pl.ds(i * 16, 16)] = zeros16
        den_v[pl.ds(i * 16, 16)] = zeros16
        return 0
    lax.fori_loop(0, (F * PST) // 16, zero_body, 0)

    bvec = bounds_v[pl.ds(wid, 16)]
    a = bvec[0]
    a_end = bvec[1]
    b = lax.bitwise_and(a, -8)
    nk = lax.shift_right_logical(a_end - b + (C - 1), LOG2C)

    nrate = nrate_v[...]
    iota16 = lax.broadcasted_iota(jnp.int32, (16,), 0)
    segb_splat = jnp.full((16,), seg_base, jnp.int32)
    ps_splat = jnp.full((16,), PS, jnp.int32)
    pst_splat = jnp.full((16,), PST, jnp.int32)
    aend_splat = jnp.full((16,), a_end, jnp.int32)
    ones16f = jnp.ones((16,), jnp.float32)
    step16 = jnp.full((16,), 16, jnp.int32)

    def dma_start(k, slot):
        start = b + lax.shift_left(k, LOG2C)
        e0 = pl.multiple_of(jnp.minimum(start, E - C), 8)
        sb = slot * C
        sem = sem0 if slot == 0 else sem1
        pltpu.async_copy(seg_hbm.at[pl.ds(e0, C)], seg_v.at[pl.ds(sb, C)],
                         sem)
        pltpu.async_copy(tin_hbm.at[pl.ds(e0, C)], tin_v.at[pl.ds(sb, C)],
                         sem)
        pltpu.async_copy(pred_hbm.at[pl.ds(e0, C)], pred_v.at[pl.ds(sb, C)],
                         sem)

    def dma_wait(slot):
        sb = slot * C
        sem = sem0 if slot == 0 else sem1
        pltpu.make_async_copy(seg_hbm.at[pl.ds(0, C)],
                              seg_v.at[pl.ds(sb, C)], sem).wait()
        pltpu.make_async_copy(tin_hbm.at[pl.ds(0, C)],
                              tin_v.at[pl.ds(sb, C)], sem).wait()
        pltpu.make_async_copy(pred_hbm.at[pl.ds(0, C)],
                              pred_v.at[pl.ds(sb, C)], sem).wait()

    def compute(k, slot):
        start = b + lax.shift_left(k, LOG2C)
        e0 = jnp.minimum(start, E - C)
        lo = jnp.maximum(a, start)
        lo_splat = jnp.full((16,), lo, jnp.int32)
        sb = slot * C
        gvv0 = jnp.full((16,), e0, jnp.int32) + iota16

        def group_body(g, gvv):
            gbase = sb + lax.shift_left(g, 4)
            segv = seg_v[pl.ds(gbase, 16)]
            tinv = tin_v[pl.ds(gbase, 16)]
            predv = pred_v[pl.ds(gbase, 16)]
            validv = jnp.logical_and(gvv >= lo_splat, gvv < aend_splat)
            slv = jnp.where(validv, segv - segb_splat, ps_splat)
            toutv = plsc.load_gather(tout_v, [slv])
            dtv = toutv - tinv
            ratev = plsc.load_gather(nrate_v, [predv])
            valv = jnp.exp(ratev * dtv)
            idxv = predv * pst_splat + slv
            plsc.addupdate_scatter(num_v, [idxv], valv, mask=validv)
            plsc.addupdate_scatter(den_v, [idxv], ones16f, mask=validv)
            return gvv + step16
        lax.fori_loop(0, C // 16, group_body, gvv0)

    @pl.when(nk > 0)
    def _():
        dma_start(0, 0)

    def pair_body(p, _):
        k0 = lax.shift_left(p, 1)
        k1 = k0 + 1

        @pl.when(k1 < nk)
        def _():
            dma_start(k1, 1)
        dma_wait(0)
        compute(k0, 0)

        @pl.when(k1 + 1 < nk)
        def _():
            dma_start(k1 + 1, 0)

        @pl.when(k1 < nk)
        def _():
            dma_wait(1)
            compute(k1, 1)
        return 0
    lax.fori_loop(0, lax.shift_right_logical(nk + 1, 1), pair_body, 0)

    for f in range(F):
        def div_body(i, _):
            o = f * PST + i * 16
            num_v[pl.ds(o, 16)] = num_v[pl.ds(o, 16)] / jnp.maximum(
                den_v[pl.ds(o, 16)], ones16f)
            return 0
        lax.fori_loop(0, PS // 16, div_body, 0)
        pltpu.sync_copy(num_v.at[pl.ds(f * PST, PS)],
                        out_hbm.at[pl.ds(f * S_PAD + seg_base, PS)])


@jax.jit
def _run(times_in, tout_pad, segment_filter_ids, oh_t, nrate, bounds):
    pred = pl.pallas_call(
        _pred_body,
        grid=(NBLKE,),
        in_specs=[pl.BlockSpec((F, BLKE), lambda i: (0, i))],
        out_specs=pl.BlockSpec((E,), lambda i: (0,)),
        out_shape=jax.ShapeDtypeStruct((E,), jnp.int32),
    )(oh_t)

    mesh = plsc.VectorSubcoreMesh(core_axis_name="c", subcore_axis_name="s")
    f = pl.kernel(
        _sc_body,
        out_type=jax.ShapeDtypeStruct((F * S_PAD,), jnp.float32),
        mesh=mesh,
        scratch_types=[
            pltpu.VMEM((PS + 16,), jnp.float32),
            pltpu.VMEM((F * PST,), jnp.float32),
            pltpu.VMEM((F * PST,), jnp.float32),
            pltpu.VMEM((2 * C,), jnp.int32),
            pltpu.VMEM((2 * C,), jnp.float32),
            pltpu.VMEM((2 * C,), jnp.int32),
            pltpu.VMEM((16,), jnp.float32),
            pltpu.VMEM((48,), jnp.int32),
            pltpu.SemaphoreType.DMA,
            pltpu.SemaphoreType.DMA,
        ],
        compiler_params=pltpu.CompilerParams(needs_layout_passes=False),
    )
    return f(times_in, tout_pad, segment_filter_ids, pred, nrate, bounds)


def kernel(times_in, times_out, segment_filter_ids, one_hot_predecessor_ids,
           decay_rate):
    nrate = -jax.nn.softplus(decay_rate)
    tout_pad = jnp.pad(times_out, (0, S_PAD - S))
    limits = jnp.minimum(jnp.arange(NW + 1, dtype=jnp.int32) * PS, S)
    bounds = jnp.searchsorted(segment_filter_ids, limits, side="left",
                              method="scan_unrolled").astype(jnp.int32)
    bounds = jnp.pad(bounds, (0, 48 - (NW + 1)))
    oh_t = one_hot_predecessor_ids.T
    out = _run(times_in, tout_pad, segment_filter_ids, oh_t, nrate, bounds)
    return out.reshape(F, S_PAD)[:, :S].T


# speedup vs baseline: 4.0288x; 1.2322x over previous
import functools

import jax
import jax.numpy as jnp
from jax import lax
from jax.experimental import pallas as pl
from jax.experimental.pallas import tpu as pltpu
from jax.experimental.pallas import tpu_sc as plsc

E = 1_600_000
S = 50_000
F = 16
NW = 32
PS = 1_568
PST = PS + 16
S_PAD = NW * PS
C = 4_096
LOG2C = 12
BLKE = 64_000
NBLKE = E // BLKE


def _pred_body(x_ref, o_ref):
    x = x_ref[...]
    w = lax.broadcasted_iota(jnp.int32, (F, 1), 0).astype(jnp.float32)
    i = pl.program_id(0)
    o_ref[pl.ds(i * BLKE, BLKE)] = jnp.sum(x * w, axis=0).astype(jnp.int32)


def _sc_body(tin_hbm, tout_hbm, seg_hbm, pred_hbm, nrate_hbm, bounds_hbm,
             out_hbm, tout_v, num_v, den_v, seg_v, tin_v, pred_v, nrate_v,
             bounds_v, sem0, sem1):
    wid = lax.axis_index("c") * 16 + lax.axis_index("s")
    seg_base = wid * PS

    pltpu.sync_copy(bounds_hbm, bounds_v)
    pltpu.sync_copy(nrate_hbm, nrate_v)
    pltpu.sync_copy(tout_hbm.at[pl.ds(seg_base, PS)], tout_v.at[pl.ds(0, PS)])
    tout_v[pl.ds(PS, 16)] = jnp.zeros((16,), jnp.float32)

    zeros16 = jnp.zeros((16,), jnp.float32)

    def zero_body(i, _):
        o = i * 64
        for u in range(4):
            num_v[pl.ds(o + u * 16, 16)] = zeros16
            den_v[pl.ds(o + u * 16, 16)] = zeros16
        return 0
    lax.fori_loop(0, (F * PST) // 64, zero_body, 0)

    bvec = bounds_v[pl.ds(wid, 16)]
    a = bvec[0]
    a_end = bvec[1]
    b = lax.bitwise_and(a, -8)
    nk = lax.shift_right_logical(a_end - b + (C - 1), LOG2C)

    nrate = nrate_v[...]
    iota16 = lax.broadcasted_iota(jnp.int32, (16,), 0)
    segb_splat = jnp.full((16,), seg_base, jnp.int32)
    ps_splat = jnp.full((16,), PS, jnp.int32)
    pst_splat = jnp.full((16,), PST, jnp.int32)
    aend_splat = jnp.full((16,), a_end, jnp.int32)
    ones16f = jnp.ones((16,), jnp.float32)
    step16 = jnp.full((16,), 16, jnp.int32)
    step32 = jnp.full((16,), 32, jnp.int32)

    def dma_start(k, slot):
        start = b + lax.shift_left(k, LOG2C)
        e0 = pl.multiple_of(jnp.minimum(start, E - C), 8)
        sb = slot * C
        sem = sem0 if slot == 0 else sem1
        pltpu.async_copy(seg_hbm.at[pl.ds(e0, C)], seg_v.at[pl.ds(sb, C)],
                         sem)
        pltpu.async_copy(tin_hbm.at[pl.ds(e0, C)], tin_v.at[pl.ds(sb, C)],
                         sem)
        pltpu.async_copy(pred_hbm.at[pl.ds(e0, C)], pred_v.at[pl.ds(sb, C)],
                         sem)

    def dma_wait(slot):
        sb = slot * C
        sem = sem0 if slot == 0 else sem1
        pltpu.make_async_copy(seg_hbm.at[pl.ds(0, C)],
                              seg_v.at[pl.ds(sb, C)], sem).wait()
        pltpu.make_async_copy(tin_hbm.at[pl.ds(0, C)],
                              tin_v.at[pl.ds(sb, C)], sem).wait()
        pltpu.make_async_copy(pred_hbm.at[pl.ds(0, C)],
                              pred_v.at[pl.ds(sb, C)], sem).wait()

    def compute(k, slot):
        start = b + lax.shift_left(k, LOG2C)
        e0 = jnp.minimum(start, E - C)
        lo = jnp.maximum(a, start)
        lo_splat = jnp.full((16,), lo, jnp.int32)
        sb = slot * C
        gvv0 = jnp.full((16,), e0, jnp.int32) + iota16

        def group_body(g, gvv):
            gbase0 = sb + lax.shift_left(g, 5)
            for u in range(2):
                gbase = gbase0 + u * 16
                segv = seg_v[pl.ds(gbase, 16)]
                tinv = tin_v[pl.ds(gbase, 16)]
                predv = pred_v[pl.ds(gbase, 16)]
                guv = gvv if u == 0 else gvv + step16
                validv = jnp.logical_and(guv >= lo_splat, guv < aend_splat)
                slv = jnp.where(validv, segv - segb_splat, ps_splat)
                toutv = plsc.load_gather(tout_v, [slv])
                dtv = toutv - tinv
                ratev = plsc.load_gather(nrate_v, [predv])
                valv = jnp.exp(ratev * dtv)
                idxv = predv * pst_splat + slv
                plsc.addupdate_scatter(num_v, [idxv], valv, mask=validv)
                plsc.addupdate_scatter(den_v, [idxv], ones16f, mask=validv)
            return gvv + step32
        lax.fori_loop(0, C // 32, group_body, gvv0)

    @pl.when(nk > 0)
    def _():
        dma_start(0, 0)

    def pair_body(p, _):
        k0 = lax.shift_left(p, 1)
        k1 = k0 + 1

        @pl.when(k1 < nk)
        def _():
            dma_start(k1, 1)
        dma_wait(0)
        compute(k0, 0)

        @pl.when(k1 + 1 < nk)
        def _():
            dma_start(k1 + 1, 0)

        @pl.when(k1 < nk)
        def _():
            dma_wait(1)
            compute(k1, 1)
        return 0
    lax.fori_loop(0, lax.shift_right_logical(nk + 1, 1), pair_body, 0)

    for f in range(F):
        def div_body(i, _):
            o = f * PST + i * 32
            for u in range(2):
                ou = o + u * 16
                num_v[pl.ds(ou, 16)] = num_v[pl.ds(ou, 16)] / jnp.maximum(
                    den_v[pl.ds(ou, 16)], ones16f)
            return 0
        lax.fori_loop(0, PS // 32, div_body, 0)
        pltpu.sync_copy(num_v.at[pl.ds(f * PST, PS)],
                        out_hbm.at[pl.ds(f * S_PAD + seg_base, PS)])


@jax.jit
def _run(times_in, tout_pad, segment_filter_ids, oh_t, nrate, bounds):
    pred = pl.pallas_call(
        _pred_body,
        grid=(NBLKE,),
        in_specs=[pl.BlockSpec((F, BLKE), lambda i: (0, i))],
        out_specs=pl.BlockSpec((E,), lambda i: (0,)),
        out_shape=jax.ShapeDtypeStruct((E,), jnp.int32),
    )(oh_t)

    mesh = plsc.VectorSubcoreMesh(core_axis_name="c", subcore_axis_name="s")
    f = pl.kernel(
        _sc_body,
        out_type=jax.ShapeDtypeStruct((F * S_PAD,), jnp.float32),
        mesh=mesh,
        scratch_types=[
            pltpu.VMEM((PS + 16,), jnp.float32),
            pltpu.VMEM((F * PST,), jnp.float32),
            pltpu.VMEM((F * PST,), jnp.float32),
            pltpu.VMEM((2 * C,), jnp.int32),
            pltpu.VMEM((2 * C,), jnp.float32),
            pltpu.VMEM((2 * C,), jnp.int32),
            pltpu.VMEM((16,), jnp.float32),
            pltpu.VMEM((48,), jnp.int32),
            pltpu.SemaphoreType.DMA,
            pltpu.SemaphoreType.DMA,
        ],
        compiler_params=pltpu.CompilerParams(needs_layout_passes=False),
    )
    return f(times_in, tout_pad, segment_filter_ids, pred, nrate, bounds)


def kernel(times_in, times_out, segment_filter_ids, one_hot_predecessor_ids,
           decay_rate):
    nrate = -jax.nn.softplus(decay_rate)
    tout_pad = jnp.pad(times_out, (0, S_PAD - S))
    limits = jnp.minimum(jnp.arange(NW + 1, dtype=jnp.int32) * PS, S)
    def bs_step(_, lh):
        lo, hi = lh
        live = lo < hi
        mid = lax.shift_right_logical(lo + hi, 1)
        go = segment_filter_ids[mid] < limits
        return (jnp.where(live & go, mid + 1, lo),
                jnp.where(live & ~go, mid, hi))
    lo0 = jnp.zeros((NW + 1,), jnp.int32)
    hi0 = jnp.full((NW + 1,), E, jnp.int32)
    bounds, _ = lax.fori_loop(0, 21, bs_step, (lo0, hi0))
    bounds = jnp.pad(bounds, (0, 48 - (NW + 1)))
    oh_t = one_hot_predecessor_ids.T
    out = _run(times_in, tout_pad, segment_filter_ids, oh_t, nrate, bounds)
    return out.reshape(F, S_PAD)[:, :S].T


# speedup vs baseline: 4.6874x; 1.1635x over previous
import functools

import jax
import jax.numpy as jnp
from jax import lax
from jax.experimental import pallas as pl
from jax.experimental.pallas import tpu as pltpu
from jax.experimental.pallas import tpu_sc as plsc

E = 1_600_000
S = 50_000
F = 16
NW = 32
PS = 1_568
PST = PS + 16
S_PAD = NW * PS
C = 8_192
LOG2C = 13
BLKE = 64_000
NBLKE = E // BLKE


def _pred_body(x_ref, o_ref):
    x = x_ref[...]
    w = lax.broadcasted_iota(jnp.int32, (F, 1), 0).astype(jnp.float32)
    i = pl.program_id(0)
    o_ref[pl.ds(i * BLKE, BLKE)] = jnp.sum(x * w, axis=0).astype(jnp.int32)


def _sc_body(tin_hbm, tout_hbm, seg_hbm, pred_hbm, nrate_hbm, bounds_hbm,
             out_hbm, tout_v, num_v, den_v, seg_v, tin_v, pred_v, nrate_v,
             bounds_v, sem0, sem1):
    wid = lax.axis_index("c") * 16 + lax.axis_index("s")
    seg_base = wid * PS

    pltpu.sync_copy(bounds_hbm, bounds_v)
    pltpu.sync_copy(nrate_hbm, nrate_v)
    pltpu.sync_copy(tout_hbm.at[pl.ds(seg_base, PS)], tout_v.at[pl.ds(0, PS)])
    tout_v[pl.ds(PS, 16)] = jnp.zeros((16,), jnp.float32)

    zeros16 = jnp.zeros((16,), jnp.float32)

    def zero_body(i, _):
        o = i * 64
        for u in range(4):
            num_v[pl.ds(o + u * 16, 16)] = zeros16
            den_v[pl.ds(o + u * 16, 16)] = zeros16
        return 0
    lax.fori_loop(0, (F * PST) // 64, zero_body, 0)

    bvec = bounds_v[pl.ds(wid, 16)]
    a = bvec[0]
    a_end = bvec[1]
    b = lax.bitwise_and(a, -8)
    nk = lax.shift_right_logical(a_end - b + (C - 1), LOG2C)

    nrate = nrate_v[...]
    iota16 = lax.broadcasted_iota(jnp.int32, (16,), 0)
    segb_splat = jnp.full((16,), seg_base, jnp.int32)
    ps_splat = jnp.full((16,), PS, jnp.int32)
    pst_splat = jnp.full((16,), PST, jnp.int32)
    aend_splat = jnp.full((16,), a_end, jnp.int32)
    ones16f = jnp.ones((16,), jnp.float32)
    step16 = jnp.full((16,), 16, jnp.int32)
    step64 = jnp.full((16,), 64, jnp.int32)

    def dma_start(k, slot):
        start = b + lax.shift_left(k, LOG2C)
        e0 = pl.multiple_of(jnp.minimum(start, E - C), 8)
        sb = slot * C
        sem = sem0 if slot == 0 else sem1
        pltpu.async_copy(seg_hbm.at[pl.ds(e0, C)], seg_v.at[pl.ds(sb, C)],
                         sem)
        pltpu.async_copy(tin_hbm.at[pl.ds(e0, C)], tin_v.at[pl.ds(sb, C)],
                         sem)
        pltpu.async_copy(pred_hbm.at[pl.ds(e0, C)], pred_v.at[pl.ds(sb, C)],
                         sem)

    def dma_wait(slot):
        sb = slot * C
        sem = sem0 if slot == 0 else sem1
        pltpu.make_async_copy(seg_hbm.at[pl.ds(0, C)],
                              seg_v.at[pl.ds(sb, C)], sem).wait()
        pltpu.make_async_copy(tin_hbm.at[pl.ds(0, C)],
                              tin_v.at[pl.ds(sb, C)], sem).wait()
        pltpu.make_async_copy(pred_hbm.at[pl.ds(0, C)],
                              pred_v.at[pl.ds(sb, C)], sem).wait()

    def compute(k, slot):
        start = b + lax.shift_left(k, LOG2C)
        e0 = jnp.minimum(start, E - C)
        lo = jnp.maximum(a, start)
        lo_splat = jnp.full((16,), lo, jnp.int32)
        sb = slot * C
        gvv0 = jnp.full((16,), e0, jnp.int32) + iota16

        def group_body(g, gvv):
            gbase0 = sb + lax.shift_left(g, 6)
            for u in range(4):
                gbase = gbase0 + u * 16
                segv = seg_v[pl.ds(gbase, 16)]
                tinv = tin_v[pl.ds(gbase, 16)]
                predv = pred_v[pl.ds(gbase, 16)]
                guv = gvv if u == 0 else gvv + jnp.full((16,), u * 16,
                                                        jnp.int32)
                validv = jnp.logical_and(guv >= lo_splat, guv < aend_splat)
                slv = jnp.where(validv, segv - segb_splat, ps_splat)
                toutv = plsc.load_gather(tout_v, [slv])
                dtv = toutv - tinv
                ratev = plsc.load_gather(nrate_v, [predv])
                valv = jnp.exp(ratev * dtv)
                idxv = predv * pst_splat + slv
                plsc.addupdate_scatter(num_v, [idxv], valv, mask=validv)
                plsc.addupdate_scatter(den_v, [idxv], ones16f, mask=validv)
            return gvv + step64
        lax.fori_loop(0, C // 64, group_body, gvv0)

    @pl.when(nk > 0)
    def _():
        dma_start(0, 0)

    def pair_body(p, _):
        k0 = lax.shift_left(p, 1)
        k1 = k0 + 1

        @pl.when(k1 < nk)
        def _():
            dma_start(k1, 1)
        dma_wait(0)
        compute(k0, 0)

        @pl.when(k1 + 1 < nk)
        def _():
            dma_start(k1 + 1, 0)

        @pl.when(k1 < nk)
        def _():
            dma_wait(1)
            compute(k1, 1)
        return 0
    lax.fori_loop(0, lax.shift_right_logical(nk + 1, 1), pair_body, 0)

    for f in range(F):
        def div_body(i, _):
            o = f * PST + i * 32
            for u in range(2):
                ou = o + u * 16
                num_v[pl.ds(ou, 16)] = num_v[pl.ds(ou, 16)] / jnp.maximum(
                    den_v[pl.ds(ou, 16)], ones16f)
            return 0
        lax.fori_loop(0, PS // 32, div_body, 0)
        pltpu.async_copy(num_v.at[pl.ds(f * PST, PS)],
                         out_hbm.at[pl.ds(f * S_PAD + seg_base, PS)], sem0)
    for f in range(F):
        pltpu.make_async_copy(
            num_v.at[pl.ds(f * PST, PS)],
            out_hbm.at[pl.ds(f * S_PAD + seg_base, PS)], sem0).wait()


@jax.jit
def _run(times_in, tout_pad, segment_filter_ids, oh_t, nrate, bounds):
    pred = pl.pallas_call(
        _pred_body,
        grid=(NBLKE,),
        in_specs=[pl.BlockSpec((F, BLKE), lambda i: (0, i))],
        out_specs=pl.BlockSpec((E,), lambda i: (0,)),
        out_shape=jax.ShapeDtypeStruct((E,), jnp.int32),
    )(oh_t)

    mesh = plsc.VectorSubcoreMesh(core_axis_name="c", subcore_axis_name="s")
    f = pl.kernel(
        _sc_body,
        out_type=jax.ShapeDtypeStruct((F * S_PAD,), jnp.float32),
        mesh=mesh,
        scratch_types=[
            pltpu.VMEM((PS + 16,), jnp.float32),
            pltpu.VMEM((F * PST,), jnp.float32),
            pltpu.VMEM((F * PST,), jnp.float32),
            pltpu.VMEM((2 * C,), jnp.int32),
            pltpu.VMEM((2 * C,), jnp.float32),
            pltpu.VMEM((2 * C,), jnp.int32),
            pltpu.VMEM((16,), jnp.float32),
            pltpu.VMEM((48,), jnp.int32),
            pltpu.SemaphoreType.DMA,
            pltpu.SemaphoreType.DMA,
        ],
        compiler_params=pltpu.CompilerParams(needs_layout_passes=False),
    )
    return f(times_in, tout_pad, segment_filter_ids, pred, nrate, bounds)


def kernel(times_in, times_out, segment_filter_ids, one_hot_predecessor_ids,
           decay_rate):
    nrate = -jax.nn.softplus(decay_rate)
    tout_pad = jnp.pad(times_out, (0, S_PAD - S))
    limits = jnp.minimum(jnp.arange(NW + 1, dtype=jnp.int32) * PS, S)
    samp = segment_filter_ids.reshape(E // 256, 256)[:, 0]
    c = jnp.sum((samp[None, :] < limits[:, None]).astype(jnp.int32), axis=1)
    s = jnp.clip((c - 1) * 256 + 1, 0, E - 256)
    widx = s[:, None] + jnp.arange(256, dtype=jnp.int32)[None, :]
    win = segment_filter_ids[widx]
    bounds = s + jnp.sum((win < limits[:, None]).astype(jnp.int32), axis=1)
    bounds = jnp.pad(bounds, (0, 48 - (NW + 1)))
    oh_t = one_hot_predecessor_ids.T
    out = _run(times_in, tout_pad, segment_filter_ids, oh_t, nrate, bounds)
    return out.reshape(F, S_PAD)[:, :S].T


# speedup vs baseline: 4.6911x; 1.0008x over previous
import jax
import jax.numpy as jnp
from jax import lax
from jax.experimental import pallas as pl
from jax.experimental.pallas import tpu as pltpu
from jax.experimental.pallas import tpu_sc as plsc

E = 1_600_000
S = 50_000
F = 16
NW = 32
PS = 1_568
PST = PS + 16
S_PAD = NW * PS
C = 8_192
LOG2C = 13
BLKE = 64_000
NBLKE = E // BLKE


def _pred_body(x_ref, o_ref):
    x = x_ref[...]
    w = lax.broadcasted_iota(jnp.int32, (F, 1), 0).astype(jnp.float32)
    i = pl.program_id(0)
    o_ref[pl.ds(i * BLKE, BLKE)] = jnp.sum(x * w, axis=0).astype(jnp.int32)


def _sc_body(tin_hbm, tout_hbm, seg_hbm, pred_hbm, nrate_hbm, bounds_hbm,
             out_hbm, tout_v, num_v, den_v, seg_v, tin_v, pred_v, nrate_v,
             bounds_v, sem0, sem1):
    wid = lax.axis_index("c") * 16 + lax.axis_index("s")
    seg_base = wid * PS

    pltpu.sync_copy(bounds_hbm, bounds_v)
    pltpu.sync_copy(nrate_hbm, nrate_v)
    pltpu.sync_copy(tout_hbm.at[pl.ds(seg_base, PS)], tout_v.at[pl.ds(0, PS)])
    tout_v[pl.ds(PS, 16)] = jnp.zeros((16,), jnp.float32)

    zeros16 = jnp.zeros((16,), jnp.float32)

    def zero_body(i, _):
        o = i * 64
        for u in range(4):
            num_v[pl.ds(o + u * 16, 16)] = zeros16
            den_v[pl.ds(o + u * 16, 16)] = zeros16
        return 0
    lax.fori_loop(0, (F * PST) // 64, zero_body, 0)

    bvec = bounds_v[pl.ds(wid, 16)]
    a = bvec[0]
    a_end = bvec[1]
    b = lax.bitwise_and(a, -8)
    nk = lax.shift_right_logical(a_end - b + (C - 1), LOG2C)

    nrate = nrate_v[...]
    iota16 = lax.broadcasted_iota(jnp.int32, (16,), 0)
    segb_splat = jnp.full((16,), seg_base, jnp.int32)
    ps_splat = jnp.full((16,), PS, jnp.int32)
    pst_splat = jnp.full((16,), PST, jnp.int32)
    aend_splat = jnp.full((16,), a_end, jnp.int32)
    ones16f = jnp.ones((16,), jnp.float32)
    step64 = jnp.full((16,), 64, jnp.int32)

    def dma_start(k, slot):
        start = b + lax.shift_left(k, LOG2C)
        e0 = pl.multiple_of(jnp.minimum(start, E - C), 8)
        sb = slot * C
        sem = sem0 if slot == 0 else sem1
        pltpu.async_copy(seg_hbm.at[pl.ds(e0, C)], seg_v.at[pl.ds(sb, C)],
                         sem)
        pltpu.async_copy(tin_hbm.at[pl.ds(e0, C)], tin_v.at[pl.ds(sb, C)],
                         sem)
        pltpu.async_copy(pred_hbm.at[pl.ds(e0, C)], pred_v.at[pl.ds(sb, C)],
                         sem)

    def dma_wait(slot):
        sb = slot * C
        sem = sem0 if slot == 0 else sem1
        pltpu.make_async_copy(seg_hbm.at[pl.ds(0, C)],
                              seg_v.at[pl.ds(sb, C)], sem).wait()
        pltpu.make_async_copy(tin_hbm.at[pl.ds(0, C)],
                              tin_v.at[pl.ds(sb, C)], sem).wait()
        pltpu.make_async_copy(pred_hbm.at[pl.ds(0, C)],
                              pred_v.at[pl.ds(sb, C)], sem).wait()

    def compute(k, slot):
        start = b + lax.shift_left(k, LOG2C)
        e0 = jnp.minimum(start, E - C)
        lo = jnp.maximum(a, start)
        lo_splat = jnp.full((16,), lo, jnp.int32)
        sb = slot * C
        gvv0 = jnp.full((16,), e0, jnp.int32) + iota16

        def group_body(g, gvv):
            gbase0 = sb + lax.shift_left(g, 6)
            for u in range(4):
                gbase = gbase0 + u * 16
                segv = seg_v[pl.ds(gbase, 16)]
                tinv = tin_v[pl.ds(gbase, 16)]
                predv = pred_v[pl.ds(gbase, 16)]
                guv = gvv if u == 0 else gvv + jnp.full((16,), u * 16,
                                                        jnp.int32)
                validv = jnp.logical_and(guv >= lo_splat, guv < aend_splat)
                slv = jnp.where(validv, segv - segb_splat, ps_splat)
                toutv = plsc.load_gather(tout_v, [slv])
                dtv = toutv - tinv
                ratev = plsc.load_gather(nrate_v, [predv])
                valv = jnp.exp(ratev * dtv)
                idxv = predv * pst_splat + slv
                plsc.addupdate_scatter(num_v, [idxv], valv, mask=validv)
                plsc.addupdate_scatter(den_v, [idxv], ones16f, mask=validv)
            return gvv + step64
        lax.fori_loop(0, C // 64, group_body, gvv0)

    @pl.when(nk > 0)
    def _():
        dma_start(0, 0)

    def pair_body(p, _):
        k0 = lax.shift_left(p, 1)
        k1 = k0 + 1

        @pl.when(k1 < nk)
        def _():
            dma_start(k1, 1)
        dma_wait(0)
        compute(k0, 0)

        @pl.when(k1 + 1 < nk)
        def _():
            dma_start(k1 + 1, 0)

        @pl.when(k1 < nk)
        def _():
            dma_wait(1)
            compute(k1, 1)
        return 0
    lax.fori_loop(0, lax.shift_right_logical(nk + 1, 1), pair_body, 0)

    for f in range(F):
        def div_body(i, _):
            o = f * PST + i * 32
            for u in range(2):
                ou = o + u * 16
                num_v[pl.ds(ou, 16)] = num_v[pl.ds(ou, 16)] / jnp.maximum(
                    den_v[pl.ds(ou, 16)], ones16f)
            return 0
        lax.fori_loop(0, PS // 32, div_body, 0)
        pltpu.async_copy(num_v.at[pl.ds(f * PST, PS)],
                         out_hbm.at[pl.ds(f * S_PAD + seg_base, PS)], sem0)
    for f in range(F):
        pltpu.make_async_copy(
            num_v.at[pl.ds(f * PST, PS)],
            out_hbm.at[pl.ds(f * S_PAD + seg_base, PS)], sem0).wait()


@jax.jit
def _run(times_in, tout_pad, segment_filter_ids, oh_t, nrate, bounds):
    pred = pl.pallas_call(
        _pred_body,
        grid=(NBLKE,),
        in_specs=[pl.BlockSpec((F, BLKE), lambda i: (0, i))],
        out_specs=pl.BlockSpec((E,), lambda i: (0,)),
        out_shape=jax.ShapeDtypeStruct((E,), jnp.int32),
    )(oh_t)

    mesh = plsc.VectorSubcoreMesh(core_axis_name="c", subcore_axis_name="s")
    f = pl.kernel(
        _sc_body,
        out_type=jax.ShapeDtypeStruct((F * S_PAD,), jnp.float32),
        mesh=mesh,
        scratch_types=[
            pltpu.VMEM((PS + 16,), jnp.float32),
            pltpu.VMEM((F * PST,), jnp.float32),
            pltpu.VMEM((F * PST,), jnp.float32),
            pltpu.VMEM((2 * C,), jnp.int32),
            pltpu.VMEM((2 * C,), jnp.float32),
            pltpu.VMEM((2 * C,), jnp.int32),
            pltpu.VMEM((16,), jnp.float32),
            pltpu.VMEM((48,), jnp.int32),
            pltpu.SemaphoreType.DMA,
            pltpu.SemaphoreType.DMA,
        ],
        compiler_params=pltpu.CompilerParams(needs_layout_passes=False),
    )
    return f(times_in, tout_pad, segment_filter_ids, pred, nrate, bounds)


def kernel(times_in, times_out, segment_filter_ids, one_hot_predecessor_ids,
           decay_rate):
    nrate = -jax.nn.softplus(decay_rate)
    tout_pad = jnp.pad(times_out, (0, S_PAD - S))
    limits = jnp.minimum(jnp.arange(NW + 1, dtype=jnp.int32) * PS, S)
    samp = segment_filter_ids.reshape(E // 256, 256)[:, 0]
    c = jnp.sum((samp[None, :] < limits[:, None]).astype(jnp.int32), axis=1)
    s = jnp.clip((c - 1) * 256 + 1, 0, E - 256)
    widx = s[:, None] + jnp.arange(256, dtype=jnp.int32)[None, :]
    win = segment_filter_ids[widx]
    bounds = s + jnp.sum((win < limits[:, None]).astype(jnp.int32), axis=1)
    bounds = jnp.pad(bounds, (0, 48 - (NW + 1)))
    oh_t = one_hot_predecessor_ids.T
    out = _run(times_in, tout_pad, segment_filter_ids, oh_t, nrate, bounds)
    return out.reshape(F, S_PAD)[:, :S].T
